# baseline (speedup 1.0000x reference)
import functools

import jax
import jax.numpy as jnp
from jax import lax
from jax.experimental import pallas as pl
from jax.experimental.pallas import tpu as pltpu
from jax.experimental.pallas import tpu_sc as plsc

N = 10000
E = 320000
D = 128
NC, NS = 2, 16
CHUNK = 128
NTILES = NC * NS
G_PER_TILE = 84
G_HALF = G_PER_TILE // 2
E_PAD = NTILES * CHUNK * G_PER_TILE
NPAD = 10240
DUMP = NPAD - 1
ROWS_PER_TILE = NPAD // NS

_mesh = plsc.VectorSubcoreMesh(core_axis_name="c", subcore_axis_name="s")


@functools.partial(
    pl.kernel,
    compiler_params=pltpu.CompilerParams(use_tc_tiling_on_sc=False),
    out_type=jax.ShapeDtypeStruct((NC, NPAD, 16), jnp.float32),
    mesh=_mesh,
    scratch_types=[
        pltpu.VMEM_SHARED((NPAD, 16), jnp.float32),
        pltpu.VMEM((CHUNK, 16), jnp.float32),
        pltpu.VMEM((CHUNK, 16), jnp.float32),
        pltpu.VMEM((G_PER_TILE, CHUNK), jnp.int32),
        pltpu.SemaphoreType.DMA,
        pltpu.SemaphoreType.DMA,
    ],
)
def _deg_call(dst_hbm, deg_out, deg_sh, ones_v, zero_v, idx_v, isem, ssem):
    cid = lax.axis_index("c")
    sid = lax.axis_index("s")
    wid = cid * NS + sid

    idx_load = pltpu.async_copy(
        dst_hbm.at[pl.ds(wid * G_PER_TILE, G_PER_TILE)], idx_v, isem)

    def fill(i, _):
        ones_v[i] = jnp.full((16,), 1.0, jnp.float32)
        zero_v[i] = jnp.zeros((16,), jnp.float32)
        return 0

    lax.fori_loop(0, CHUNK, fill, 0)
    def zinit(r, _):
        pltpu.sync_copy(zero_v, deg_sh.at[pl.ds(sid * ROWS_PER_TILE + r * CHUNK, CHUNK)])
        return 0

    lax.fori_loop(0, ROWS_PER_TILE // CHUNK, zinit, 0)
    idx_load.wait()
    plsc.subcore_barrier()

    def body(g, _):
        pltpu.async_copy(ones_v, deg_sh.at[idx_v.at[g]], ssem, add=True)
        return 0

    lax.fori_loop(0, G_PER_TILE, body, 0)

    def drain(g, _):
        pltpu.make_async_copy(ones_v, deg_sh.at[idx_v.at[0]], ssem).wait()
        return 0

    lax.fori_loop(0, G_PER_TILE, drain, 0)
    plsc.subcore_barrier()
    pltpu.sync_copy(
        deg_sh.at[pl.ds(sid * ROWS_PER_TILE, ROWS_PER_TILE)],
        deg_out.at[cid, pl.ds(sid * ROWS_PER_TILE, ROWS_PER_TILE)],
    )


def _sigma_body(w_ref, o_ref):
    w = w_ref[...]
    m = jnp.dot(w, w.T, preferred_element_type=jnp.float32)

    def it(_, v):
        u = jnp.dot(v, m, preferred_element_type=jnp.float32)
        nrm = jnp.sqrt(jnp.sum(u * u))
        return u / nrm

    v = lax.fori_loop(0, 48, it, jnp.full((1, D), 1.0 / (D ** 0.5), jnp.float32))
    u = jnp.dot(v, m, preferred_element_type=jnp.float32)
    lam = jnp.sqrt(jnp.sum(u * u))
    o_ref[...] = jnp.full((1, 1), jnp.sqrt(lam), jnp.float32)


def _sigma_call(w):
    return pl.pallas_call(
        _sigma_body,
        out_shape=jax.ShapeDtypeStruct((1, 1), jnp.float32),
    )(w)


_RB = 512


def _xs_body(x_ref, w_ref, sig_ref, deg_ref, o_ref):
    inv_sigma = 1.0 / sig_ref[0, 0]
    d = deg_ref[0, :, 0:1] + deg_ref[1, :, 0:1] + 1.0
    dinv = lax.rsqrt(d)
    xw = lax.dot_general(
        x_ref[...], w_ref[...], (((1,), (1,)), ((), ())),
        preferred_element_type=jnp.float32,
    )
    o_ref[...] = xw * (dinv * inv_sigma)


def _xs_call(x_pad, w, sig, deg):
    return pl.pallas_call(
        _xs_body,
        grid=(NPAD // _RB,),
        in_specs=[
            pl.BlockSpec((_RB, D), lambda i: (i, 0)),
            pl.BlockSpec((D, D), lambda i: (0, 0)),
            pl.BlockSpec((1, 1), lambda i: (0, 0)),
            pl.BlockSpec((NC, _RB, 16), lambda i: (0, i, 0)),
        ],
        out_specs=pl.BlockSpec((_RB, D), lambda i: (i, 0)),
        out_shape=jax.ShapeDtypeStruct((NPAD, D), jnp.float32),
    )(x_pad, w, sig, deg)


@functools.partial(
    pl.kernel,
    compiler_params=pltpu.CompilerParams(use_tc_tiling_on_sc=False),
    out_type=jax.ShapeDtypeStruct((NC, NPAD, D), jnp.float32),
    mesh=_mesh,
    scratch_types=[
        pltpu.VMEM_SHARED((NPAD, D), jnp.float32),
        pltpu.VMEM((CHUNK, D), jnp.float32),
        pltpu.VMEM((CHUNK, D), jnp.float32),
        pltpu.VMEM((G_HALF, CHUNK), jnp.int32),
        pltpu.VMEM((G_HALF, CHUNK), jnp.int32),
        pltpu.SemaphoreType.DMA,
        pltpu.SemaphoreType.DMA,
        pltpu.SemaphoreType.DMA,
    ],
)
def _acc_call(xs_hbm, src_hbm, dst_hbm, acc_out,
              acc_sh, rows_a, rows_b, src_v, dst_v, sem_a, sem_b, isem):
    cid = lax.axis_index("c")
    sid = lax.axis_index("s")
    wid = cid * NS + sid

    src_load = pltpu.async_copy(
        src_hbm.at[pl.ds(wid * G_PER_TILE, G_HALF)], src_v, isem)
    dst_load = pltpu.async_copy(
        dst_hbm.at[pl.ds(wid * G_PER_TILE, G_HALF)], dst_v, isem)

    def zfill(k, _):
        rows_a[k // (D // 16), pl.ds((k % (D // 16)) * 16, 16)] = jnp.zeros(
            (16,), jnp.float32)
        return 0

    lax.fori_loop(0, CHUNK * (D // 16), zfill, 0)

    def zinit(r, _):
        pltpu.sync_copy(rows_a, acc_sh.at[pl.ds(sid * ROWS_PER_TILE + r * CHUNK, CHUNK)])
        return 0

    lax.fori_loop(0, ROWS_PER_TILE // CHUNK, zinit, 0)
    src_load.wait()
    dst_load.wait()
    plsc.subcore_barrier()

    for h in range(2):
        pltpu.async_copy(xs_hbm.at[src_v.at[0]], rows_a, sem_a)

        def body(g2, _):
            g = g2 * 2
            pltpu.make_async_copy(xs_hbm.at[src_v.at[g]], rows_a, sem_a).wait()
            pltpu.async_copy(xs_hbm.at[src_v.at[g + 1]], rows_b, sem_b)
            pltpu.sync_copy(rows_a, acc_sh.at[dst_v.at[g]], add=True)

            @pl.when(g2 < G_HALF // 2 - 1)
            def _():
                pltpu.async_copy(xs_hbm.at[src_v.at[g + 2]], rows_a, sem_a)

            pltpu.make_async_copy(xs_hbm.at[src_v.at[g + 1]], rows_b, sem_b).wait()
            pltpu.sync_copy(rows_b, acc_sh.at[dst_v.at[g + 1]], add=True)
            return 0

        lax.fori_loop(0, G_HALF // 2, body, 0)
        if h == 0:
            src_load2 = pltpu.async_copy(
                src_hbm.at[pl.ds(wid * G_PER_TILE + G_HALF, G_HALF)], src_v, isem)
            dst_load2 = pltpu.async_copy(
                dst_hbm.at[pl.ds(wid * G_PER_TILE + G_HALF, G_HALF)], dst_v, isem)
            src_load2.wait()
            dst_load2.wait()
    plsc.subcore_barrier()
    pltpu.sync_copy(
        acc_sh.at[pl.ds(sid * ROWS_PER_TILE, ROWS_PER_TILE)],
        acc_out.at[cid, pl.ds(sid * ROWS_PER_TILE, ROWS_PER_TILE)],
    )


_FB = 400


def _fin_body(acc_ref, xs_ref, deg_ref, b_ref, o_ref):
    d = deg_ref[0, :, 0:1] + deg_ref[1, :, 0:1] + 1.0
    dinv = lax.rsqrt(d)
    y = (acc_ref[0] + acc_ref[1] + xs_ref[...]) * dinv + b_ref[...]
    mu = jnp.mean(y, axis=1, keepdims=True)
    yc = y - mu
    var = jnp.mean(yc * yc, axis=1, keepdims=True)
    o = yc * lax.rsqrt(var + 1e-5)
    o_ref[...] = jnp.where(o >= 0.0, o, 0.2 * o)


def _fin_call(acc, xs, deg, b2):
    return pl.pallas_call(
        _fin_body,
        grid=(N // _FB,),
        in_specs=[
            pl.BlockSpec((NC, _FB, D), lambda i: (0, i, 0)),
            pl.BlockSpec((_FB, D), lambda i: (i, 0)),
            pl.BlockSpec((NC, _FB, 16), lambda i: (0, i, 0)),
            pl.BlockSpec((1, D), lambda i: (0, 0)),
        ],
        out_specs=pl.BlockSpec((_FB, D), lambda i: (i, 0)),
        out_shape=jax.ShapeDtypeStruct((N, D), jnp.float32),
    )(acc, xs, deg, b2)


def kernel(x, edge_index, W, b):
    src = edge_index[0].astype(jnp.int32)
    dst = edge_index[1].astype(jnp.int32)
    pad = jnp.full((E_PAD - E,), DUMP, jnp.int32)
    src_p = jnp.concatenate([src, pad]).reshape(NTILES * G_PER_TILE, CHUNK)
    dst_p = jnp.concatenate([dst, pad]).reshape(NTILES * G_PER_TILE, CHUNK)
    x_p = jnp.pad(x, ((0, NPAD - N), (0, 0)))

    deg = _deg_call(dst_p)
    sig = _sigma_call(W)
    xs = _xs_call(x_p, W, sig, deg)
    acc = _acc_call(xs, src_p, dst_p)
    return _fin_call(acc, xs, deg, b.reshape(1, D))


# speedup vs baseline: 4.7367x; 4.7367x over previous
import functools

import jax
import jax.numpy as jnp
from jax import lax
from jax.experimental import pallas as pl
from jax.experimental.pallas import tpu as pltpu
from jax.experimental.pallas import tpu_sc as plsc

N = 10000
E = 320000
D = 128
NC, NS = 2, 16
CHUNK = 128
NTILES = NC * NS
G_PER_TILE = 84
G_HALF = G_PER_TILE // 2
E_PAD = NTILES * CHUNK * G_PER_TILE
NPAD = 10240
DUMP = NPAD - 1
ROWS_PER_TILE = NPAD // NS

_mesh = plsc.VectorSubcoreMesh(core_axis_name="c", subcore_axis_name="s")


@functools.partial(
    pl.kernel,
    compiler_params=pltpu.CompilerParams(use_tc_tiling_on_sc=False),
    out_type=jax.ShapeDtypeStruct((NC, NPAD, 16), jnp.float32),
    mesh=_mesh,
    scratch_types=[
        pltpu.VMEM_SHARED((NPAD, 16), jnp.float32),
        pltpu.VMEM((CHUNK, 16), jnp.float32),
        pltpu.VMEM((CHUNK, 16), jnp.float32),
        pltpu.VMEM((G_PER_TILE, CHUNK), jnp.int32),
        pltpu.SemaphoreType.DMA,
        pltpu.SemaphoreType.DMA,
    ],
)
def _deg_call(dst_hbm, deg_out, deg_sh, ones_v, zero_v, idx_v, isem, ssem):
    cid = lax.axis_index("c")
    sid = lax.axis_index("s")
    wid = cid * NS + sid

    idx_load = pltpu.async_copy(
        dst_hbm.at[pl.ds(wid * G_PER_TILE, G_PER_TILE)], idx_v, isem)

    def fill(i, _):
        ones_v[i] = jnp.full((16,), 1.0, jnp.float32)
        zero_v[i] = jnp.zeros((16,), jnp.float32)
        return 0

    lax.fori_loop(0, CHUNK, fill, 0)
    def zinit(r, _):
        pltpu.sync_copy(zero_v, deg_sh.at[pl.ds(sid * ROWS_PER_TILE + r * CHUNK, CHUNK)])
        return 0

    lax.fori_loop(0, ROWS_PER_TILE // CHUNK, zinit, 0)
    idx_load.wait()
    plsc.subcore_barrier()

    def body(g, _):
        pltpu.async_copy(ones_v, deg_sh.at[idx_v.at[g]], ssem, add=True)
        return 0

    lax.fori_loop(0, G_PER_TILE, body, 0)

    def drain(g, _):
        pltpu.make_async_copy(ones_v, deg_sh.at[idx_v.at[0]], ssem).wait()
        return 0

    lax.fori_loop(0, G_PER_TILE, drain, 0)
    plsc.subcore_barrier()
    pltpu.sync_copy(
        deg_sh.at[pl.ds(sid * ROWS_PER_TILE, ROWS_PER_TILE)],
        deg_out.at[cid, pl.ds(sid * ROWS_PER_TILE, ROWS_PER_TILE)],
    )


def _sigma_body(w_ref, o_ref):
    w = w_ref[...]
    m = jnp.dot(w, w.T, preferred_element_type=jnp.float32)

    def it(_, v):
        u = jnp.dot(v, m, preferred_element_type=jnp.float32)
        nrm = jnp.sqrt(jnp.sum(u * u))
        return u / nrm

    v = lax.fori_loop(0, 48, it, jnp.full((1, D), 1.0 / (D ** 0.5), jnp.float32))
    u = jnp.dot(v, m, preferred_element_type=jnp.float32)
    lam = jnp.sqrt(jnp.sum(u * u))
    o_ref[...] = jnp.full((1, 1), jnp.sqrt(lam), jnp.float32)


def _sigma_call(w):
    return pl.pallas_call(
        _sigma_body,
        out_shape=jax.ShapeDtypeStruct((1, 1), jnp.float32),
    )(w)


_RB = 512


def _xs_body(x_ref, w_ref, sig_ref, deg_ref, o_ref):
    inv_sigma = 1.0 / sig_ref[0, 0]
    d = deg_ref[0, :, 0:1] + deg_ref[1, :, 0:1] + 1.0
    dinv = lax.rsqrt(d)
    xw = lax.dot_general(
        x_ref[...], w_ref[...], (((1,), (1,)), ((), ())),
        preferred_element_type=jnp.float32,
    )
    o_ref[...] = xw * (dinv * inv_sigma)


def _xs_call(x_pad, w, sig, deg):
    return pl.pallas_call(
        _xs_body,
        grid=(NPAD // _RB,),
        in_specs=[
            pl.BlockSpec((_RB, D), lambda i: (i, 0)),
            pl.BlockSpec((D, D), lambda i: (0, 0)),
            pl.BlockSpec((1, 1), lambda i: (0, 0)),
            pl.BlockSpec((NC, _RB, 16), lambda i: (0, i, 0)),
        ],
        out_specs=pl.BlockSpec((_RB, D), lambda i: (i, 0)),
        out_shape=jax.ShapeDtypeStruct((NPAD, D), jnp.float32),
    )(x_pad, w, sig, deg)


DH = D // NC
G_TILE_ALL = NTILES * G_PER_TILE // NS
N_STAGE = 4
G_STAGE = G_TILE_ALL // N_STAGE


@functools.partial(
    pl.kernel,
    compiler_params=pltpu.CompilerParams(use_tc_tiling_on_sc=False),
    out_type=jax.ShapeDtypeStruct((NPAD, D), jnp.float32),
    mesh=_mesh,
    scratch_types=[
        pltpu.VMEM_SHARED((NPAD, DH), jnp.float32),
        pltpu.VMEM_SHARED((NPAD, DH), jnp.float32),
        pltpu.VMEM((CHUNK, DH), jnp.float32),
        pltpu.VMEM((CHUNK, DH), jnp.float32),
        pltpu.VMEM((G_STAGE, CHUNK), jnp.int32),
        pltpu.VMEM((G_STAGE, CHUNK), jnp.int32),
        pltpu.SemaphoreType.DMA,
        pltpu.SemaphoreType.DMA,
        pltpu.SemaphoreType.DMA,
    ],
)
def _acc_call(xs_hbm, src_hbm, dst_hbm, acc_out,
              xs_sh, acc_sh, rows_a, rows_b, src_v, dst_v, sem_a, sem_b, isem):
    cid = lax.axis_index("c")
    sid = lax.axis_index("s")
    tile_c0 = sid * G_TILE_ALL

    src_load = pltpu.async_copy(
        src_hbm.at[pl.ds(tile_c0, G_STAGE)], src_v, isem)
    dst_load = pltpu.async_copy(
        dst_hbm.at[pl.ds(tile_c0, G_STAGE)], dst_v, isem)

    pltpu.sync_copy(
        xs_hbm.at[pl.ds(sid * ROWS_PER_TILE, ROWS_PER_TILE), pl.ds(cid * DH, DH)],
        xs_sh.at[pl.ds(sid * ROWS_PER_TILE, ROWS_PER_TILE)],
    )

    def zfill(k, _):
        rows_a[k // (DH // 16), pl.ds((k % (DH // 16)) * 16, 16)] = jnp.zeros(
            (16,), jnp.float32)
        return 0

    lax.fori_loop(0, CHUNK * (DH // 16), zfill, 0)

    def zinit(r, _):
        pltpu.sync_copy(rows_a, acc_sh.at[pl.ds(sid * ROWS_PER_TILE + r * CHUNK, CHUNK)])
        return 0

    lax.fori_loop(0, ROWS_PER_TILE // CHUNK, zinit, 0)
    src_load.wait()
    dst_load.wait()
    plsc.subcore_barrier()

    for h in range(N_STAGE):
        pltpu.async_copy(xs_sh.at[src_v.at[0]], rows_a, sem_a)

        def body(g2, _):
            g = g2 * 2
            pltpu.make_async_copy(xs_sh.at[src_v.at[g]], rows_a, sem_a).wait()
            pltpu.async_copy(xs_sh.at[src_v.at[g + 1]], rows_b, sem_b)
            pltpu.sync_copy(rows_a, acc_sh.at[dst_v.at[g]], add=True)

            @pl.when(g2 < G_STAGE // 2 - 1)
            def _():
                pltpu.async_copy(xs_sh.at[src_v.at[g + 2]], rows_a, sem_a)

            pltpu.make_async_copy(xs_sh.at[src_v.at[g + 1]], rows_b, sem_b).wait()
            pltpu.sync_copy(rows_b, acc_sh.at[dst_v.at[g + 1]], add=True)
            return 0

        lax.fori_loop(0, G_STAGE // 2, body, 0)
        if h < N_STAGE - 1:
            nxt = tile_c0 + (h + 1) * G_STAGE
            src_load2 = pltpu.async_copy(src_hbm.at[pl.ds(nxt, G_STAGE)], src_v, isem)
            dst_load2 = pltpu.async_copy(dst_hbm.at[pl.ds(nxt, G_STAGE)], dst_v, isem)
            src_load2.wait()
            dst_load2.wait()
    plsc.subcore_barrier()
    pltpu.sync_copy(
        acc_sh.at[pl.ds(sid * ROWS_PER_TILE, ROWS_PER_TILE)],
        acc_out.at[pl.ds(sid * ROWS_PER_TILE, ROWS_PER_TILE), pl.ds(cid * DH, DH)],
    )


_FB = 400


def _fin_body(acc_ref, xs_ref, deg_ref, b_ref, o_ref):
    d = deg_ref[0, :, 0:1] + deg_ref[1, :, 0:1] + 1.0
    dinv = lax.rsqrt(d)
    y = (acc_ref[...] + xs_ref[...]) * dinv + b_ref[...]
    mu = jnp.mean(y, axis=1, keepdims=True)
    yc = y - mu
    var = jnp.mean(yc * yc, axis=1, keepdims=True)
    o = yc * lax.rsqrt(var + 1e-5)
    o_ref[...] = jnp.where(o >= 0.0, o, 0.2 * o)


def _fin_call(acc, xs, deg, b2):
    return pl.pallas_call(
        _fin_body,
        grid=(N // _FB,),
        in_specs=[
            pl.BlockSpec((_FB, D), lambda i: (i, 0)),
            pl.BlockSpec((_FB, D), lambda i: (i, 0)),
            pl.BlockSpec((NC, _FB, 16), lambda i: (0, i, 0)),
            pl.BlockSpec((1, D), lambda i: (0, 0)),
        ],
        out_specs=pl.BlockSpec((_FB, D), lambda i: (i, 0)),
        out_shape=jax.ShapeDtypeStruct((N, D), jnp.float32),
    )(acc, xs, deg, b2)


def kernel(x, edge_index, W, b):
    src = edge_index[0].astype(jnp.int32)
    dst = edge_index[1].astype(jnp.int32)
    pad = jnp.full((E_PAD - E,), DUMP, jnp.int32)
    src_p = jnp.concatenate([src, pad]).reshape(NTILES * G_PER_TILE, CHUNK)
    dst_p = jnp.concatenate([dst, pad]).reshape(NTILES * G_PER_TILE, CHUNK)
    x_p = jnp.pad(x, ((0, NPAD - N), (0, 0)))

    deg = _deg_call(dst_p)
    sig = _sigma_call(W)
    xs = _xs_call(x_p, W, sig, deg)
    acc = _acc_call(xs, src_p, dst_p)
    return _fin_call(acc, xs, deg, b.reshape(1, D))
